# baseline (speedup 1.0000x reference)
import jax
import jax.numpy as jnp
from jax import lax
from jax.experimental import pallas as pl
from jax.experimental.pallas import tpu as pltpu
from jax.experimental.pallas import tpu_sc as plsc

L = 16
NQ = 4096
NR = 8192
KTOP = 8
NW = 32
QPW = NQ // NW
NCHUNK = NR // L
NQB = 8
PREF = 64
MARGIN = 0.01


def _dist(qxv, qyv, qzv, rxv, ryv, rzv):
    dx = qxv - rxv
    dy = qyv - ryv
    dz = qzv - rzv
    d = dx * dx + dy * dy
    return d + dz * dz


def _cmetric(qxv, qyv, qzv, rxv, ryv, rzv, r2hv):
    s = qxv * rxv + qyv * ryv
    s = s + qzv * rzv
    return r2hv - s


def _knn_body(rx_h, ry_h, rz_h, qx_h, qy_h, qz_h, out_h,
              rx, ry, rz, qx, qy, qz, ci0, ci1, ci2, ci3, ci4, ci5, ci6,
              ci7, outb):
    cid = lax.axis_index("c")
    sid = lax.axis_index("s")
    wid = sid * 2 + cid
    qbase = wid * QPW

    pltpu.sync_copy(rx_h, rx)
    pltpu.sync_copy(ry_h, ry)
    pltpu.sync_copy(rz_h, rz)
    pltpu.sync_copy(qx_h.at[pl.ds(qbase, QPW)], qx)
    pltpu.sync_copy(qy_h.at[pl.ds(qbase, QPW)], qy)
    pltpu.sync_copy(qz_h.at[pl.ds(qbase, QPW)], qz)

    ci = (ci0, ci1, ci2, ci3, ci4, ci5, ci6, ci7)

    iota = lax.iota(jnp.int32, L)
    inf16 = jnp.full((L,), jnp.inf, jnp.float32)
    zeros16i = jnp.zeros((L,), jnp.int32)

    def group_body(g, carry):
        gq = g * NQB
        blk = (gq // L) * L
        qx16 = qx[pl.ds(blk, L)]
        qy16 = qy[pl.ds(blk, L)]
        qz16 = qz[pl.ds(blk, L)]
        qsx, qsy, qsz = [], [], []
        for j in range(NQB):
            lane = jnp.full((L,), gq - blk + j, jnp.int32)
            qsx.append(jnp.take_along_axis(qx16, lane, axis=0,
                                           mode="promise_in_bounds"))
            qsy.append(jnp.take_along_axis(qy16, lane, axis=0,
                                           mode="promise_in_bounds"))
            qsz.append(jnp.take_along_axis(qz16, lane, axis=0,
                                           mode="promise_in_bounds"))

        def p1_body(c, Ms):
            base = c * L
            rxv = rx[pl.ds(base, L)]
            ryv = ry[pl.ds(base, L)]
            rzv = rz[pl.ds(base, L)]
            return tuple(
                jnp.minimum(Ms[j],
                            _dist(qsx[j], qsy[j], qsz[j], rxv, ryv, rzv))
                for j in range(NQB))

        Ms = plsc.parallel_loop(0, PREF, unroll=2, carry=(inf16,) * NQB)(
            lambda c, Ms: p1_body(c, Ms))
        ts = []
        for j in range(NQB):
            srt = lax.sort(Ms[j], dimension=0)
            ts.append(srt[KTOP - 1])

        def p2_body(c, offs):
            base = c * L
            rxv = rx[pl.ds(base, L)]
            ryv = ry[pl.ds(base, L)]
            rzv = rz[pl.ds(base, L)]
            idxv = base + iota
            new_offs = []
            for j in range(NQB):
                cm = _dist(qsx[j], qsy[j], qsz[j], rxv, ryv, rzv)
                m = cm <= ts[j]
                pos = offs[j] + plsc.cumsum(m.astype(jnp.int32)) - 1
                plsc.store_scatter(ci[j], [pos], idxv, mask=m)
                pc = plsc.all_reduce_population_count(m)
                new_offs.append(offs[j] + pc)
            return tuple(new_offs)

        offv = plsc.parallel_loop(0, NCHUNK, unroll=4,
                                  carry=(zeros16i,) * NQB)(
            lambda c, offs: p2_body(c, offs))
        offs = [offv[j][0] for j in range(NQB)]

        for j in range(NQB):
            n = offs[j]
            nch = (n + L - 1) // L

            def p3_body(k, st, j=j, n=n):
                B, Bi = st
                base = k * L
                iv_raw = ci[j][pl.ds(base, L)]
                valid = (base + iota) < n
                iv = jnp.where(valid, iv_raw, 0)
                gx = plsc.load_gather(rx, [iv])
                gy = plsc.load_gather(ry, [iv])
                gz = plsc.load_gather(rz, [iv])
                d = _dist(qsx[j], qsy[j], qsz[j], gx, gy, gz)
                d = jnp.where(valid, d, jnp.inf)
                d_asc, i_asc = plsc.sort_key_val(d, iv)
                m = d_asc < B
                nB = jnp.where(m, d_asc, B)
                nBi = jnp.where(m, i_asc, Bi)
                nB, nBi = plsc.sort_key_val(nB, nBi, descending=True)
                return nB, nBi

            B, Bi = lax.fori_loop(0, nch, p3_body, (inf16, zeros16i))
            _, i_fin = plsc.sort_key_val(B, Bi)
            outb[gq + j, :] = i_fin
        return carry

    lax.fori_loop(0, QPW // NQB, group_body, 0)
    pltpu.sync_copy(outb, out_h.at[pl.ds(qbase, QPW)])


@jax.jit
def _knn(rx, ry, rz, qx, qy, qz):
    mesh = plsc.VectorSubcoreMesh(core_axis_name="c", subcore_axis_name="s",
                                  num_cores=2, num_subcores=16)
    return pl.kernel(
        _knn_body,
        out_type=jax.ShapeDtypeStruct((NQ, L), jnp.int32),
        mesh=mesh,
        compiler_params=pltpu.CompilerParams(needs_layout_passes=False),
        scratch_types=[
            pltpu.VMEM((NR,), jnp.float32),
            pltpu.VMEM((NR,), jnp.float32),
            pltpu.VMEM((NR,), jnp.float32),
            pltpu.VMEM((QPW,), jnp.float32),
            pltpu.VMEM((QPW,), jnp.float32),
            pltpu.VMEM((QPW,), jnp.float32),
        ] + [pltpu.VMEM((NR,), jnp.int32)] * NQB + [
            pltpu.VMEM((QPW, L), jnp.int32),
        ],
    )(rx, ry, rz, qx, qy, qz)


def kernel(query, reference_pts):
    q = jnp.asarray(query, jnp.float32)
    r = jnp.asarray(reference_pts, jnp.float32)
    qx, qy, qz = q[:, 0], q[:, 1], q[:, 2]
    rx, ry, rz = r[:, 0], r[:, 1], r[:, 2]
    out = _knn(rx, ry, rz, qx, qy, qz)
    return out[:, :KTOP]


# speedup vs baseline: 2.2121x; 2.2121x over previous
import jax
import jax.numpy as jnp
from jax import lax
from jax.experimental import pallas as pl
from jax.experimental.pallas import tpu as pltpu
from jax.experimental.pallas import tpu_sc as plsc

L = 16
NQ = 4096
NR = 8192
KTOP = 8
NW = 32
QPW = NQ // NW
NCHUNK = NR // L
NQB = 8
PREF = 64
MARGIN = 0.01


def _dist(qxv, qyv, qzv, rxv, ryv, rzv):
    dx = qxv - rxv
    dy = qyv - ryv
    dz = qzv - rzv
    d = dx * dx + dy * dy
    return d + dz * dz


def _cmetric(qxv, qyv, qzv, rxv, ryv, rzv, r2hv):
    s = qxv * rxv + qyv * ryv
    s = s + qzv * rzv
    return r2hv - s


def _knn_body(rx_h, ry_h, rz_h, qx_h, qy_h, qz_h, out_h,
              rx, ry, rz, qx, qy, qz, ci0, ci1, ci2, ci3, ci4, ci5, ci6,
              ci7, outb):
    cid = lax.axis_index("c")
    sid = lax.axis_index("s")
    wid = sid * 2 + cid
    qbase = wid * QPW

    pltpu.sync_copy(rx_h, rx)
    pltpu.sync_copy(ry_h, ry)
    pltpu.sync_copy(rz_h, rz)
    pltpu.sync_copy(qx_h.at[pl.ds(qbase, QPW)], qx)
    pltpu.sync_copy(qy_h.at[pl.ds(qbase, QPW)], qy)
    pltpu.sync_copy(qz_h.at[pl.ds(qbase, QPW)], qz)

    ci = (ci0, ci1, ci2, ci3, ci4, ci5, ci6, ci7)

    iota = lax.iota(jnp.int32, L)
    inf16 = jnp.full((L,), jnp.inf, jnp.float32)
    zeros16i = jnp.zeros((L,), jnp.int32)

    def group_body(g, carry):
        gq = g * NQB
        blk = (gq // L) * L
        qx16 = qx[pl.ds(blk, L)]
        qy16 = qy[pl.ds(blk, L)]
        qz16 = qz[pl.ds(blk, L)]
        qsx, qsy, qsz = [], [], []
        for j in range(NQB):
            lane = jnp.full((L,), gq - blk + j, jnp.int32)
            qsx.append(jnp.take_along_axis(qx16, lane, axis=0,
                                           mode="promise_in_bounds"))
            qsy.append(jnp.take_along_axis(qy16, lane, axis=0,
                                           mode="promise_in_bounds"))
            qsz.append(jnp.take_along_axis(qz16, lane, axis=0,
                                           mode="promise_in_bounds"))

        def p1_body(c, Ms):
            base = c * L
            rxv = rx[pl.ds(base, L)]
            ryv = ry[pl.ds(base, L)]
            rzv = rz[pl.ds(base, L)]
            return tuple(
                jnp.minimum(Ms[j],
                            _dist(qsx[j], qsy[j], qsz[j], rxv, ryv, rzv))
                for j in range(NQB))

        Ms = plsc.parallel_loop(0, PREF, unroll=2, carry=(inf16,) * NQB)(
            lambda c, Ms: p1_body(c, Ms))
        ts = []
        for j in range(NQB):
            srt = lax.sort(Ms[j], dimension=0)
            ts.append(srt[KTOP - 1])

        def p2_body(c, offs):
            base = c * L
            rxv = rx[pl.ds(base, L)]
            ryv = ry[pl.ds(base, L)]
            rzv = rz[pl.ds(base, L)]
            idxv = base + iota
            new_offs = []
            for j in range(NQB):
                cm = _dist(qsx[j], qsy[j], qsz[j], rxv, ryv, rzv)
                m = cm <= ts[j]
                pos = offs[j] + plsc.cumsum(m.astype(jnp.int32)) - 1
                plsc.store_scatter(ci[j], [pos], idxv, mask=m)
                pc = plsc.all_reduce_population_count(m)
                new_offs.append(offs[j] + pc)
            return tuple(new_offs)

        offv = plsc.parallel_loop(0, NCHUNK, unroll=2,
                                  carry=(zeros16i,) * NQB)(
            lambda c, offs: p2_body(c, offs))
        offs = [offv[j][0] for j in range(NQB)]

        for j in range(NQB):
            n = offs[j]
            nch = (n + L - 1) // L

            def p3_body(k, st, j=j, n=n):
                B, Bi = st
                base = k * L
                iv_raw = ci[j][pl.ds(base, L)]
                valid = (base + iota) < n
                iv = jnp.where(valid, iv_raw, 0)
                gx = plsc.load_gather(rx, [iv])
                gy = plsc.load_gather(ry, [iv])
                gz = plsc.load_gather(rz, [iv])
                d = _dist(qsx[j], qsy[j], qsz[j], gx, gy, gz)
                d = jnp.where(valid, d, jnp.inf)
                d_asc, i_asc = plsc.sort_key_val(d, iv)
                m = d_asc < B
                nB = jnp.where(m, d_asc, B)
                nBi = jnp.where(m, i_asc, Bi)
                nB, nBi = plsc.sort_key_val(nB, nBi, descending=True)
                return nB, nBi

            B, Bi = lax.fori_loop(0, nch, p3_body, (inf16, zeros16i))
            _, i_fin = plsc.sort_key_val(B, Bi)
            outb[gq + j, :] = i_fin
        return carry

    lax.fori_loop(0, QPW // NQB, group_body, 0)
    pltpu.sync_copy(outb, out_h.at[pl.ds(qbase, QPW)])


@jax.jit
def _knn(rx, ry, rz, qx, qy, qz):
    mesh = plsc.VectorSubcoreMesh(core_axis_name="c", subcore_axis_name="s",
                                  num_cores=2, num_subcores=16)
    return pl.kernel(
        _knn_body,
        out_type=jax.ShapeDtypeStruct((NQ, L), jnp.int32),
        mesh=mesh,
        compiler_params=pltpu.CompilerParams(needs_layout_passes=False),
        scratch_types=[
            pltpu.VMEM((NR,), jnp.float32),
            pltpu.VMEM((NR,), jnp.float32),
            pltpu.VMEM((NR,), jnp.float32),
            pltpu.VMEM((QPW,), jnp.float32),
            pltpu.VMEM((QPW,), jnp.float32),
            pltpu.VMEM((QPW,), jnp.float32),
        ] + [pltpu.VMEM((NR,), jnp.int32)] * NQB + [
            pltpu.VMEM((QPW, L), jnp.int32),
        ],
    )(rx, ry, rz, qx, qy, qz)


def kernel(query, reference_pts):
    q = jnp.asarray(query, jnp.float32)
    r = jnp.asarray(reference_pts, jnp.float32)
    qx, qy, qz = q[:, 0], q[:, 1], q[:, 2]
    rx, ry, rz = r[:, 0], r[:, 1], r[:, 2]
    out = _knn(rx, ry, rz, qx, qy, qz)
    return out[:, :KTOP]


# speedup vs baseline: 2.2181x; 1.0027x over previous
import jax
import jax.numpy as jnp
from jax import lax
from jax.experimental import pallas as pl
from jax.experimental.pallas import tpu as pltpu
from jax.experimental.pallas import tpu_sc as plsc

L = 16
NQ = 4096
NR = 8192
KTOP = 8
NW = 32
QPW = NQ // NW
NCHUNK = NR // L
NQB = 8
PREF = 64
MARGIN = 0.01


def _dist(qxv, qyv, qzv, rxv, ryv, rzv):
    dx = qxv - rxv
    dy = qyv - ryv
    dz = qzv - rzv
    d = dx * dx + dy * dy
    return d + dz * dz


def _cmetric(qxv, qyv, qzv, rxv, ryv, rzv, r2hv):
    s = qxv * rxv + qyv * ryv
    s = s + qzv * rzv
    return r2hv - s


def _knn_body(rx_h, ry_h, rz_h, qx_h, qy_h, qz_h, out_h,
              rx, ry, rz, qx, qy, qz, ci0, ci1, ci2, ci3, ci4, ci5, ci6,
              ci7, outb):
    cid = lax.axis_index("c")
    sid = lax.axis_index("s")
    wid = sid * 2 + cid
    qbase = wid * QPW

    pltpu.sync_copy(rx_h, rx)
    pltpu.sync_copy(ry_h, ry)
    pltpu.sync_copy(rz_h, rz)
    pltpu.sync_copy(qx_h.at[pl.ds(qbase, QPW)], qx)
    pltpu.sync_copy(qy_h.at[pl.ds(qbase, QPW)], qy)
    pltpu.sync_copy(qz_h.at[pl.ds(qbase, QPW)], qz)

    ci = (ci0, ci1, ci2, ci3, ci4, ci5, ci6, ci7)

    iota = lax.iota(jnp.int32, L)
    inf16 = jnp.full((L,), jnp.inf, jnp.float32)
    zeros16i = jnp.zeros((L,), jnp.int32)

    def group_body(g, carry):
        gq = g * NQB
        blk = (gq // L) * L
        qx16 = qx[pl.ds(blk, L)]
        qy16 = qy[pl.ds(blk, L)]
        qz16 = qz[pl.ds(blk, L)]
        qsx, qsy, qsz = [], [], []
        for j in range(NQB):
            lane = jnp.full((L,), gq - blk + j, jnp.int32)
            qsx.append(jnp.take_along_axis(qx16, lane, axis=0,
                                           mode="promise_in_bounds"))
            qsy.append(jnp.take_along_axis(qy16, lane, axis=0,
                                           mode="promise_in_bounds"))
            qsz.append(jnp.take_along_axis(qz16, lane, axis=0,
                                           mode="promise_in_bounds"))

        def p1_body(c, Ms):
            base = c * L
            rxv = rx[pl.ds(base, L)]
            ryv = ry[pl.ds(base, L)]
            rzv = rz[pl.ds(base, L)]
            return tuple(
                jnp.minimum(Ms[j],
                            _dist(qsx[j], qsy[j], qsz[j], rxv, ryv, rzv))
                for j in range(NQB))

        Ms = lax.fori_loop(0, PREF, p1_body, (inf16,) * NQB)
        ts = []
        for j in range(NQB):
            srt = lax.sort(Ms[j], dimension=0)
            ts.append(srt[KTOP - 1])

        def p2_body(c, offs):
            base = c * L
            rxv = rx[pl.ds(base, L)]
            ryv = ry[pl.ds(base, L)]
            rzv = rz[pl.ds(base, L)]
            idxv = base + iota
            new_offs = []
            for j in range(NQB):
                cm = _dist(qsx[j], qsy[j], qsz[j], rxv, ryv, rzv)
                m = cm <= ts[j]
                pos = offs[j] + plsc.cumsum(m.astype(jnp.int32)) - 1
                plsc.store_scatter(ci[j], [pos], idxv, mask=m)
                pc = plsc.all_reduce_population_count(m)
                new_offs.append(offs[j] + pc)
            return tuple(new_offs)

        offv = plsc.parallel_loop(0, NCHUNK, unroll=2,
                                  carry=(zeros16i,) * NQB)(
            lambda c, offs: p2_body(c, offs))
        offs = [offv[j][0] for j in range(NQB)]

        for j in range(NQB):
            n = offs[j]
            nch = (n + L - 1) // L

            def p3_body(k, st, j=j, n=n):
                B, Bi = st
                base = k * L
                iv_raw = ci[j][pl.ds(base, L)]
                valid = (base + iota) < n
                iv = jnp.where(valid, iv_raw, 0)
                gx = plsc.load_gather(rx, [iv])
                gy = plsc.load_gather(ry, [iv])
                gz = plsc.load_gather(rz, [iv])
                d = _dist(qsx[j], qsy[j], qsz[j], gx, gy, gz)
                d = jnp.where(valid, d, jnp.inf)
                d_asc, i_asc = plsc.sort_key_val(d, iv)
                m = d_asc < B
                nB = jnp.where(m, d_asc, B)
                nBi = jnp.where(m, i_asc, Bi)
                nB, nBi = plsc.sort_key_val(nB, nBi, descending=True)
                return nB, nBi

            B, Bi = lax.fori_loop(0, nch, p3_body, (inf16, zeros16i))
            _, i_fin = plsc.sort_key_val(B, Bi)
            outb[gq + j, :] = i_fin
        return carry

    lax.fori_loop(0, QPW // NQB, group_body, 0)
    pltpu.sync_copy(outb, out_h.at[pl.ds(qbase, QPW)])


@jax.jit
def _knn(rx, ry, rz, qx, qy, qz):
    mesh = plsc.VectorSubcoreMesh(core_axis_name="c", subcore_axis_name="s",
                                  num_cores=2, num_subcores=16)
    return pl.kernel(
        _knn_body,
        out_type=jax.ShapeDtypeStruct((NQ, L), jnp.int32),
        mesh=mesh,
        compiler_params=pltpu.CompilerParams(needs_layout_passes=False),
        scratch_types=[
            pltpu.VMEM((NR,), jnp.float32),
            pltpu.VMEM((NR,), jnp.float32),
            pltpu.VMEM((NR,), jnp.float32),
            pltpu.VMEM((QPW,), jnp.float32),
            pltpu.VMEM((QPW,), jnp.float32),
            pltpu.VMEM((QPW,), jnp.float32),
        ] + [pltpu.VMEM((NR,), jnp.int32)] * NQB + [
            pltpu.VMEM((QPW, L), jnp.int32),
        ],
    )(rx, ry, rz, qx, qy, qz)


def kernel(query, reference_pts):
    q = jnp.asarray(query, jnp.float32)
    r = jnp.asarray(reference_pts, jnp.float32)
    qx, qy, qz = q[:, 0], q[:, 1], q[:, 2]
    rx, ry, rz = r[:, 0], r[:, 1], r[:, 2]
    out = _knn(rx, ry, rz, qx, qy, qz)
    return out[:, :KTOP]
